# baseline (speedup 1.0000x reference)
import jax
import jax.numpy as jnp
from jax.experimental import pallas as pl

HIDDEN = 2048
NUM_EXPERTS = 8
TOP_K = 2
BLOCK = 2048


def _gate_block(x_ref, w_ref, logits_ref):
    logits_ref[...] = jnp.dot(x_ref[...], w_ref[...],
                              preferred_element_type=jnp.float32)


def kernel(hidden_states, W_gate):
    B, S, H = hidden_states.shape
    T = B * S
    x = hidden_states.reshape(T, H)
    grid = (T // BLOCK,)

    logits = pl.pallas_call(
        _gate_block,
        grid=grid,
        in_specs=[
            pl.BlockSpec((BLOCK, H), lambda i: (i, 0)),
            pl.BlockSpec((H, NUM_EXPERTS), lambda i: (0, 0)),
        ],
        out_specs=pl.BlockSpec((BLOCK, NUM_EXPERTS), lambda i: (i, 0)),
        out_shape=jax.ShapeDtypeStruct((T, NUM_EXPERTS), jnp.float32),
    )(x, W_gate)

    logits = logits.reshape(B, S, NUM_EXPERTS)
    tw, ti = jax.lax.top_k(logits, TOP_K)
    rw = jax.nn.softmax(tw, axis=-1)
    return (rw, ti, logits)
